# baseline (speedup 1.0000x reference)
import jax
import jax.numpy as jnp
from jax.experimental import pallas as pl

N = 10000
D_IN = 256
D_OUT = 256
BM = 480


def _fused_kernel(f_ref, w_ref, adj_ref, o_ref):
    agg = jnp.dot(adj_ref[...], f_ref[...],
                  preferred_element_type=jnp.float32)
    o_ref[...] = jnp.dot(agg, w_ref[...],
                         preferred_element_type=jnp.float32)


def kernel(features, adj, weight):
    return pl.pallas_call(
        _fused_kernel,
        grid=(pl.cdiv(N, BM),),
        in_specs=[
            pl.BlockSpec((N, D_IN), lambda i: (0, 0)),
            pl.BlockSpec((D_IN, D_OUT), lambda i: (0, 0)),
            pl.BlockSpec((BM, N), lambda i: (i, 0)),
        ],
        out_specs=pl.BlockSpec((BM, D_OUT), lambda i: (i, 0)),
        out_shape=jax.ShapeDtypeStruct((N, D_OUT), jnp.float32),
    )(features, weight, adj)


# speedup vs baseline: 1.0049x; 1.0049x over previous
import jax
import jax.numpy as jnp
from jax.experimental import pallas as pl

N = 10000
D_IN = 256
D_OUT = 256
BM = 400


def _fused_kernel(f_ref, w_ref, adj_ref, o_ref):
    agg = jnp.dot(adj_ref[...], f_ref[...],
                  preferred_element_type=jnp.float32)
    o_ref[...] = jnp.dot(agg, w_ref[...],
                         preferred_element_type=jnp.float32)


def kernel(features, adj, weight):
    return pl.pallas_call(
        _fused_kernel,
        grid=(pl.cdiv(N, BM),),
        in_specs=[
            pl.BlockSpec((N, D_IN), lambda i: (0, 0)),
            pl.BlockSpec((D_IN, D_OUT), lambda i: (0, 0)),
            pl.BlockSpec((BM, N), lambda i: (i, 0)),
        ],
        out_specs=pl.BlockSpec((BM, D_OUT), lambda i: (i, 0)),
        out_shape=jax.ShapeDtypeStruct((N, D_OUT), jnp.float32),
    )(features, weight, adj)


# speedup vs baseline: 1.0053x; 1.0003x over previous
import jax
import jax.numpy as jnp
from jax.experimental import pallas as pl
import jax.experimental.pallas.tpu as pltpu

N = 10000
D_IN = 256
D_OUT = 256
BM = 400


def _fused_kernel(f_ref, w_ref, adj_ref, o_ref):
    agg = jnp.dot(adj_ref[...], f_ref[...],
                  preferred_element_type=jnp.float32)
    o_ref[...] = jnp.dot(agg, w_ref[...],
                         preferred_element_type=jnp.float32)


def kernel(features, adj, weight):
    return pl.pallas_call(
        _fused_kernel,
        grid=(pl.cdiv(N, BM),),
        in_specs=[
            pl.BlockSpec((N, D_IN), lambda i: (0, 0)),
            pl.BlockSpec((D_IN, D_OUT), lambda i: (0, 0)),
            pl.BlockSpec((BM, N), lambda i: (i, 0)),
        ],
        out_specs=pl.BlockSpec((BM, D_OUT), lambda i: (i, 0)),
        out_shape=jax.ShapeDtypeStruct((N, D_OUT), jnp.float32),
        compiler_params=pltpu.CompilerParams(
            dimension_semantics=(pltpu.PARALLEL,)),
    )(features, weight, adj)
